# Initial kernel scaffold; baseline (speedup 1.0000x reference)
import jax
import jax.numpy as jnp
from jax.experimental import pallas as pl


def kernel(idx, betas, global_orient, body_pose, transl):
    raise NotImplementedError("write your pallas kernel here")


# speedup vs baseline: 2.2759x; 2.2759x over previous
import functools

import jax
import jax.numpy as jnp
from jax import lax
from jax.experimental import pallas as pl
from jax.experimental.pallas import tpu as pltpu
from jax.experimental.pallas import tpu_sc as plsc

_NC = 2
_NS = 16
_NW = _NC * _NS


def _gather3_sc(idx, go, bp, tr):
    B = idx.shape[0]
    b_per_w = B // _NW
    d_go, d_bp, d_tr = go.shape[1], bp.shape[1], tr.shape[1]
    mesh = plsc.VectorSubcoreMesh(core_axis_name="c", subcore_axis_name="s")

    @functools.partial(
        pl.kernel,
        mesh=mesh,
        out_type=(
            jax.ShapeDtypeStruct((B, d_go), go.dtype),
            jax.ShapeDtypeStruct((B, d_bp), bp.dtype),
            jax.ShapeDtypeStruct((B, d_tr), tr.dtype),
        ),
        scratch_types=[
            pltpu.VMEM((b_per_w,), jnp.int32),
            pltpu.VMEM((b_per_w, d_go), go.dtype),
            pltpu.VMEM((b_per_w, d_bp), bp.dtype),
            pltpu.VMEM((b_per_w, d_tr), tr.dtype),
            pltpu.SemaphoreType.DMA,
        ],
    )
    def k(go_hbm, bp_hbm, tr_hbm, idx_hbm, ogo_hbm, obp_hbm, otr_hbm,
          idx_v, go_v, bp_v, tr_v, sem):
        wid = lax.axis_index("s") * _NC + lax.axis_index("c")
        base = wid * b_per_w
        pltpu.sync_copy(idx_hbm.at[pl.ds(base, b_per_w)], idx_v)

        @pl.loop(0, b_per_w, step=16)
        def _(c):
            v = idx_v[pl.ds(c, 16)]
            for k in range(16):
                j = v[k]
                pltpu.async_copy(go_hbm.at[pl.ds(j, 1)], go_v.at[pl.ds(c + k, 1)], sem)
                pltpu.async_copy(bp_hbm.at[pl.ds(j, 1)], bp_v.at[pl.ds(c + k, 1)], sem)
                pltpu.async_copy(tr_hbm.at[pl.ds(j, 1)], tr_v.at[pl.ds(c + k, 1)], sem)

        @pl.loop(0, b_per_w)
        def _(i):
            pltpu.make_async_copy(go_hbm.at[pl.ds(0, 1)], go_v.at[pl.ds(i, 1)], sem).wait()
            pltpu.make_async_copy(bp_hbm.at[pl.ds(0, 1)], bp_v.at[pl.ds(i, 1)], sem).wait()
            pltpu.make_async_copy(tr_hbm.at[pl.ds(0, 1)], tr_v.at[pl.ds(i, 1)], sem).wait()

        pltpu.sync_copy(go_v, ogo_hbm.at[pl.ds(base, b_per_w)])
        pltpu.sync_copy(bp_v, obp_hbm.at[pl.ds(base, b_per_w)])
        pltpu.sync_copy(tr_v, otr_hbm.at[pl.ds(base, b_per_w)])

    return k(go, bp, tr, idx)


def _betas_broadcast_tc(betas, B):
    d = betas.shape[1]

    def body(b_ref, o_ref):
        o_ref[...] = jnp.broadcast_to(b_ref[0:1, :], o_ref.shape)

    return pl.pallas_call(
        body,
        out_shape=jax.ShapeDtypeStruct((B, d), betas.dtype),
        grid=(1,),
        in_specs=[pl.BlockSpec((8, d), lambda i: (0, 0))],
        out_specs=pl.BlockSpec((B, d), lambda i: (0, 0)),
    )(betas)


def kernel(idx, betas, global_orient, body_pose, transl):
    B = idx.shape[0]
    idx = idx.astype(jnp.int32)
    out_betas = _betas_broadcast_tc(betas, B)
    out_go, out_bp, out_tr = _gather3_sc(idx, global_orient, body_pose, transl)
    return (out_betas, out_go, out_bp, out_tr)
